# Initial kernel scaffold; baseline (speedup 1.0000x reference)
import jax
import jax.numpy as jnp
from jax.experimental import pallas as pl


def kernel(x_user, x_item, W_u2i_src, W_u2i_tgt, W_i2u_src, W_i2u_tgt, edge_index_u2i, edge_index_i2u):
    raise NotImplementedError("write your pallas kernel here")


# speedup vs baseline: 2.0713x; 2.0713x over previous
import functools

import jax
import jax.numpy as jnp
from jax import lax
from jax.experimental import pallas as pl
from jax.experimental.pallas import tpu as pltpu
from jax.experimental.pallas import tpu_sc as plsc

N = 10000
D = 128
E = 320000
NC = 2
NS = 16
K = 64
NCHUNK = 320
EPT = NCHUNK * K
PAD = NS * EPT - E
NP = 10240
RPT = NP // NS


def _dot(a, b):
    return lax.dot_general(a, b, (((1,), (0,)), ((), ())),
                           preferred_element_type=jnp.float32,
                           precision=lax.Precision.HIGHEST)


def _proj_body(x_ref, w_ref, o_ref):
    o_ref[...] = _dot(x_ref[...], w_ref[0])


def _src_projections(x_cat, w_stack):
    nb = N // 1000
    return pl.pallas_call(
        _proj_body,
        grid=(2, nb),
        in_specs=[
            pl.BlockSpec((1000, D), lambda j, i: (j * nb + i, 0)),
            pl.BlockSpec((1, D, D), lambda j, i: (j, 0, 0)),
        ],
        out_specs=pl.BlockSpec((1000, D), lambda j, i: (j * nb + i, 0)),
        out_shape=jax.ShapeDtypeStruct((2 * N, D), jnp.float32),
    )(x_cat, w_stack)


def _combine_body(x_ref, w_ref, s_ref, c_ref, o_ref):
    xw = _dot(x_ref[...], w_ref[0])
    cnt = c_ref[0][:, 0:1]
    mean = s_ref[0] / jnp.maximum(cnt, 1.0)
    o_ref[0] = jnp.maximum(xw + mean, 0.0)


def _combine(x_cat, wt_stack, sums, cnts):
    nb = N // 1000
    return pl.pallas_call(
        _combine_body,
        grid=(2, nb),
        in_specs=[
            pl.BlockSpec((1000, D), lambda j, i: ((1 - j) * nb + i, 0)),
            pl.BlockSpec((1, D, D), lambda j, i: (j, 0, 0)),
            pl.BlockSpec((1, 1000, D), lambda j, i: (j, i, 0)),
            pl.BlockSpec((1, 1000, 16), lambda j, i: (j, i, 0)),
        ],
        out_specs=pl.BlockSpec((1, 1000, D), lambda j, i: (j, i, 0)),
        out_shape=jax.ShapeDtypeStruct((2, N, D), jnp.float32),
    )(x_cat, wt_stack, sums, cnts)


def _sc_body(srcx, idx_h, zr_h,
             sums_o,
             idx_v, rows0_v, rows1_v, acc_sh,
             isem0, isem1, isem2, isem3, gsem0, gsem1):
    c = lax.axis_index("c")
    s = lax.axis_index("s")
    wid = c * NS + s
    isems = (isem0, isem1, isem2, isem3)
    gsems = (gsem0, gsem1)
    rows = (rows0_v, rows1_v)

    def fetch_idx(t, nb):
        pltpu.async_copy(idx_h.at[wid, t], idx_v.at[nb], isems[nb])

    def wait_idx(t, nb):
        pltpu.make_async_copy(idx_h.at[wid, t], idx_v.at[nb], isems[nb]).wait()

    def gather(nb, rb):
        pltpu.async_copy(srcx.at[idx_v.at[nb, 0]], rows[rb], gsems[rb])

    def wait_gather(nb, rb):
        pltpu.make_async_copy(srcx.at[idx_v.at[nb, 0]], rows[rb],
                              gsems[rb]).wait()

    def scatter(nb, rb):
        pltpu.sync_copy(rows[rb], acc_sh.at[idx_v.at[nb, 1]], add=True)

    pltpu.sync_copy(zr_h, acc_sh.at[pl.ds(s * RPT, RPT)])
    plsc.subcore_barrier()

    def step(jj, carry):
        j = jj * 2
        fetch_idx(j, 0)
        fetch_idx(j + 1, 1)
        wait_idx(j, 0)
        gather(0, 0)
        wait_idx(j + 1, 1)
        gather(1, 1)
        wait_gather(0, 0)
        scatter(0, 0)
        wait_gather(1, 1)
        scatter(1, 1)
        return carry

    lax.fori_loop(0, NCHUNK // 2, step, 0)
    plsc.subcore_barrier()

    for r in range(RPT // K):
        base = s * RPT + r * K
        pltpu.sync_copy(acc_sh.at[pl.ds(base, K)], rows0_v)
        pltpu.sync_copy(rows0_v, sums_o.at[c, pl.ds(base, K)])


@functools.cache
def _sc_aggregate():
    return pl.kernel(
        _sc_body,
        out_type=jax.ShapeDtypeStruct((2, NP, D), jnp.float32),
        mesh=plsc.VectorSubcoreMesh(core_axis_name="c", subcore_axis_name="s",
                                    num_cores=NC, num_subcores=NS),
        scratch_types=[
            pltpu.VMEM((2, 2, K), jnp.int32),
            pltpu.VMEM((K, D), jnp.float32),
            pltpu.VMEM((K, D), jnp.float32),
            pltpu.VMEM_SHARED((NP, D), jnp.float32),
            pltpu.SemaphoreType.DMA,
            pltpu.SemaphoreType.DMA,
            pltpu.SemaphoreType.DMA,
            pltpu.SemaphoreType.DMA,
            pltpu.SemaphoreType.DMA,
            pltpu.SemaphoreType.DMA,
        ],
    )


@jax.jit
def kernel(x_user, x_item, W_u2i_src, W_u2i_tgt, W_i2u_src, W_i2u_tgt,
           edge_index_u2i, edge_index_i2u):
    x_cat = jnp.concatenate([x_user, x_item], axis=0)
    w_src = jnp.stack([W_u2i_src, W_i2u_src])
    w_tgt = jnp.stack([W_u2i_tgt, W_i2u_tgt])

    srcx = _src_projections(x_cat, w_src)

    spad = jnp.zeros((PAD,), jnp.int32)
    dpad = jnp.full((PAD,), N, jnp.int32)
    sidx = jnp.concatenate([
        jnp.concatenate([edge_index_u2i[0], spad]).reshape(NS, NCHUNK, K),
        jnp.concatenate([edge_index_i2u[0] + N, spad]).reshape(NS, NCHUNK, K),
    ], axis=0)
    didx = jnp.concatenate([
        jnp.concatenate([edge_index_u2i[1], dpad]).reshape(NS, NCHUNK, K),
        jnp.concatenate([edge_index_i2u[1], dpad]).reshape(NS, NCHUNK, K),
    ], axis=0)
    idx = jnp.stack([sidx, didx], axis=2)

    zr = jnp.zeros((RPT, D), jnp.float32)

    sums = _sc_aggregate()(srcx, idx, zr)
    ones_e = jnp.ones((E,), jnp.float32)
    cnt2 = jnp.stack([
        jax.ops.segment_sum(ones_e, edge_index_u2i[1], num_segments=NP),
        jax.ops.segment_sum(ones_e, edge_index_i2u[1], num_segments=NP),
    ])
    cnts = jnp.broadcast_to(cnt2[:, :, None], (2, NP, 16))
    out = _combine(x_cat, w_tgt, sums, cnts)
    return out[1], out[0]
